# Initial kernel scaffold; baseline (speedup 1.0000x reference)
import jax
import jax.numpy as jnp
from jax.experimental import pallas as pl


def kernel(R, idx_i, idx_j, pair_mask, z, point_mask):
    raise NotImplementedError("write your pallas kernel here")


# speedup vs baseline: 4.9334x; 4.9334x over previous
import functools

import jax
import jax.numpy as jnp
import numpy as np
from jax import lax
from jax.experimental import pallas as pl
from jax.experimental.pallas import tpu as pltpu
from jax.experimental.pallas import tpu_sc as plsc

N_NODES = 100000
N_PAIRS = 3200000
R_CUT = 5.0
N_RBF = 16
LAMBDA = 1.0

NC = 2
NS = 16
L = 16
NW = NC * NS
PPW = N_PAIRS // NW
B = 800
G = B // L
NBLK = PPW // B

_centers = np.linspace(0.0, R_CUT, N_RBF).astype(np.float32)
_delta = np.float32(_centers[1] - _centers[0])
_GAMMA = np.float32(np.float32(0.5) / (_delta * _delta))
_NEG_GAMMA = np.float32(-_GAMMA)

_xs = 0.5 * (1.0 + np.cos(np.pi * (np.arange(512) + 0.5) / 512.0))
_ys = 0.5 * (1.0 + np.cos(np.pi * _xs))
_COS_COEF = np.polyfit(_xs, _ys, 12).astype(np.float32)

_SPH_C = np.float32(0.4886025119029199)
_L2C1 = np.float32(1.0925484305920792)
_L2C2 = np.float32(0.31539156525252005)
_L2C3 = np.float32(0.5462742152960396)

_RSQRT_MAGIC = np.int32(0x5F3759DF)


def _sc_body(rpad_hbm, idxi_hbm, idxj_hbm, pm_hbm, zblk_hbm,
             r_hbm, u_hbm, d_hbm, rbf_hbm, phi_hbm, sph_hbm, chi_hbm,
             idxi_v, idxj_v, pm_v, ri_v, rj_v,
             r_v, u_v, d_v, rbf_v, phi_v, sph_v, w_v, chi_sh, sem):
    cid = lax.axis_index("c")
    sid = lax.axis_index("s")
    wid = cid * NS + sid

    nchunks = N_NODES // B
    full_rounds = nchunks // NS
    pltpu.sync_copy(zblk_hbm, w_v)
    for t in range(full_rounds):
        r0 = pl.multiple_of((sid + NS * t) * B, B)
        pltpu.sync_copy(w_v, chi_sh.at[pl.ds(r0, B)])
    tail_cidx = sid + NS * full_rounds

    @pl.when(tail_cidx < nchunks)
    def _zero_tail():
        r0 = pl.multiple_of(tail_cidx * B, B)
        pltpu.sync_copy(w_v, chi_sh.at[pl.ds(r0, B)])

    plsc.subcore_barrier()

    iota = lax.iota(jnp.int32, L)

    def do_block(blk, _):
        base = wid * PPW + blk * B

        pltpu.sync_copy(idxi_hbm.at[pl.ds(base, B)], idxi_v)
        pltpu.sync_copy(idxj_hbm.at[pl.ds(base, B)], idxj_v)
        pltpu.sync_copy(pm_hbm.at[pl.ds(base, B)], pm_v)

        cp1 = pltpu.async_copy(rpad_hbm.at[idxi_v], ri_v, sem)
        cp2 = pltpu.async_copy(rpad_hbm.at[idxj_v], rj_v, sem)
        cp1.wait()
        cp2.wait()

        c0 = jnp.zeros((L,), jnp.int32)
        c1 = jnp.full((L,), 1, jnp.int32)
        c2 = jnp.full((L,), 2, jnp.int32)

        def do_group(g, _):
            p = g * L + iota
            off = g * L

            xi = plsc.load_gather(ri_v, [p, c0])
            yi = plsc.load_gather(ri_v, [p, c1])
            zi = plsc.load_gather(ri_v, [p, c2])
            xj = plsc.load_gather(rj_v, [p, c0])
            yj = plsc.load_gather(rj_v, [p, c1])
            zj = plsc.load_gather(rj_v, [p, c2])
            pm = pm_v[pl.ds(off, L)]

            rx = (xj - xi) * pm
            ry = (yj - yi) * pm
            rz = (zj - zi) * pm

            d2 = rx * rx + ry * ry + rz * rz
            safe = d2 > 0.0
            d2s = jnp.where(safe, d2, 1.0)
            bits = lax.bitcast_convert_type(d2s, jnp.int32)
            y = lax.bitcast_convert_type(
                _RSQRT_MAGIC - lax.shift_right_logical(bits, 1), jnp.float32)
            for _i in range(3):
                y = y * (1.5 - 0.5 * d2s * y * y)
            ds_ = d2s * y
            d = jnp.where(safe, ds_, 0.0) * pm

            idx3 = p * 3
            plsc.store_scatter(r_v, [idx3], rx)
            plsc.store_scatter(r_v, [idx3 + 1], ry)
            plsc.store_scatter(r_v, [idx3 + 2], rz)
            d_v[pl.ds(off, L)] = d

            idx16 = p * 16
            for k in range(N_RBF):
                t = d - float(_centers[k])
                e = jnp.exp(_NEG_GAMMA * (t * t)) * pm
                plsc.store_scatter(rbf_v, [idx16 + k], e)

            s = d * np.float32(1.0 / R_CUT)
            acc = jnp.full((L,), float(_COS_COEF[0]), jnp.float32)
            for ck in _COS_COEF[1:]:
                acc = acc * s + float(ck)
            phi = jnp.where(d < R_CUT, acc, 0.0) * pm
            phi_v[pl.ds(off, L)] = phi

            nz = d != 0.0
            inv = 1.0 / jnp.where(nz, d, 1.0)
            ux = jnp.where(nz, rx * inv, 0.0) * pm
            uy = jnp.where(nz, ry * inv, 0.0) * pm
            uz = jnp.where(nz, rz * inv, 0.0) * pm
            plsc.store_scatter(u_v, [idx3], ux)
            plsc.store_scatter(u_v, [idx3 + 1], uy)
            plsc.store_scatter(u_v, [idx3 + 2], uz)

            s0 = _SPH_C * uy * pm
            s1 = _SPH_C * uz * pm
            s2 = _SPH_C * ux * pm
            s3 = _L2C1 * ux * uy * pm
            s4 = _L2C1 * uy * uz * pm
            s5 = _L2C2 * (3.0 * uz * uz - 1.0) * pm
            s6 = _L2C1 * ux * uz * pm
            s7 = _L2C3 * (ux * ux - uy * uy) * pm

            idx8 = p * 8
            for k, sv in enumerate((s0, s1, s2, s3, s4, s5, s6, s7)):
                plsc.store_scatter(sph_v, [idx8 + k], sv)
                plsc.store_scatter(w_v, [p, jnp.full((L,), k, jnp.int32)], sv * phi)
            return 0

        lax.fori_loop(0, G, do_group, 0, unroll=False)

        pltpu.sync_copy(r_v, r_hbm.at[pl.ds(base * 3, B * 3)])
        pltpu.sync_copy(u_v, u_hbm.at[pl.ds(base * 3, B * 3)])
        pltpu.sync_copy(d_v, d_hbm.at[pl.ds(base, B)])
        pltpu.sync_copy(rbf_v, rbf_hbm.at[pl.ds(base * 16, B * 16)])
        pltpu.sync_copy(phi_v, phi_hbm.at[pl.ds(base, B)])
        pltpu.sync_copy(sph_v, sph_hbm.at[pl.ds(base * 8, B * 8)])

        pltpu.sync_copy(w_v, chi_sh.at[idxi_v], add=True)
        return 0

    lax.fori_loop(0, NBLK, do_block, 0, unroll=False)

    plsc.subcore_barrier()

    for t in range(full_rounds):
        r0 = pl.multiple_of((sid + NS * t) * B, B)
        pltpu.sync_copy(chi_sh.at[pl.ds(r0, B)], w_v)
        pltpu.sync_copy(w_v, chi_hbm.at[cid, pl.ds(r0, B)])

    @pl.when(tail_cidx < nchunks)
    def _write_tail():
        r0 = pl.multiple_of(tail_cidx * B, B)
        pltpu.sync_copy(chi_sh.at[pl.ds(r0, B)], w_v)
        pltpu.sync_copy(w_v, chi_hbm.at[cid, pl.ds(r0, B)])


_sc_call = pl.kernel(
    _sc_body,
    out_type=(
        jax.ShapeDtypeStruct((N_PAIRS * 3,), jnp.float32),
        jax.ShapeDtypeStruct((N_PAIRS * 3,), jnp.float32),
        jax.ShapeDtypeStruct((N_PAIRS,), jnp.float32),
        jax.ShapeDtypeStruct((N_PAIRS * 16,), jnp.float32),
        jax.ShapeDtypeStruct((N_PAIRS,), jnp.float32),
        jax.ShapeDtypeStruct((N_PAIRS * 8,), jnp.float32),
        jax.ShapeDtypeStruct((NC, N_NODES, 8), jnp.float32),
    ),
    mesh=plsc.VectorSubcoreMesh(core_axis_name="c", subcore_axis_name="s"),
    compiler_params=pltpu.CompilerParams(
        needs_layout_passes=False, use_tc_tiling_on_sc=False),
    scratch_types=(
        pltpu.VMEM((B,), jnp.int32),
        pltpu.VMEM((B,), jnp.int32),
        pltpu.VMEM((B,), jnp.float32),
        pltpu.VMEM((B, 8), jnp.float32),
        pltpu.VMEM((B, 8), jnp.float32),
        pltpu.VMEM((B * 3,), jnp.float32),
        pltpu.VMEM((B * 3,), jnp.float32),
        pltpu.VMEM((B,), jnp.float32),
        pltpu.VMEM((B * 16,), jnp.float32),
        pltpu.VMEM((B,), jnp.float32),
        pltpu.VMEM((B * 8,), jnp.float32),
        pltpu.VMEM((B, 8), jnp.float32),
        pltpu.VMEM_SHARED((N_NODES, 8), jnp.float32),
        pltpu.SemaphoreType.DMA,
    ),
)


def _combine_body(p_ref, m_ref, o_ref):
    o_ref[...] = (p_ref[0] + p_ref[1]) * m_ref[...] * np.float32(1.0 / LAMBDA)


_combine = pl.pallas_call(
    _combine_body,
    out_shape=jax.ShapeDtypeStruct((N_NODES * 8 // 128, 128), jnp.float32),
)


def kernel(R, idx_i, idx_j, pair_mask, z, point_mask):
    del z
    rpad = jnp.zeros((N_NODES, 8), jnp.float32).at[:, :3].set(R)
    zblk = jnp.zeros((B, 8), jnp.float32)
    r_f, u_f, d_ij, rbf_f, phi, sph_f, chi_parts = _sc_call(
        rpad, idx_i, idx_j, pair_mask, zblk)
    mask_rep = jnp.repeat(point_mask, 8).reshape(N_NODES * 8 // 128, 128)
    chi = _combine(chi_parts.reshape(NC, N_NODES * 8 // 128, 128), mask_rep)
    return (
        r_f.reshape(N_PAIRS, 3),
        u_f.reshape(N_PAIRS, 3),
        d_ij,
        rbf_f.reshape(N_PAIRS, 16),
        phi,
        sph_f.reshape(N_PAIRS, 8),
        chi.reshape(N_NODES, 8),
    )


# speedup vs baseline: 5.7973x; 1.1751x over previous
import functools

import jax
import jax.numpy as jnp
import numpy as np
from jax import lax
from jax.experimental import pallas as pl
from jax.experimental.pallas import tpu as pltpu
from jax.experimental.pallas import tpu_sc as plsc

N_NODES = 100000
N_PAIRS = 3200000
R_CUT = 5.0
N_RBF = 16
LAMBDA = 1.0

NC = 2
NS = 16
L = 16
NW = NC * NS
PPW = N_PAIRS // NW
B = 800
G = B // L
NBLK = PPW // B

_centers = np.linspace(0.0, R_CUT, N_RBF).astype(np.float32)
_delta = np.float32(_centers[1] - _centers[0])
_GAMMA = np.float32(np.float32(0.5) / (_delta * _delta))
_NEG_GAMMA = np.float32(-_GAMMA)

_xs = 0.5 * (1.0 + np.cos(np.pi * (np.arange(512) + 0.5) / 512.0))
_ys = 0.5 * (1.0 + np.cos(np.pi * _xs))
_COS_COEF = np.polyfit(_xs, _ys, 12).astype(np.float32)

_SPH_C = np.float32(0.4886025119029199)
_L2C1 = np.float32(1.0925484305920792)
_L2C2 = np.float32(0.31539156525252005)
_L2C3 = np.float32(0.5462742152960396)

_RSQRT_MAGIC = np.int32(0x5F3759DF)


def _sc_body(rpad_hbm, idxi_hbm, idxj_hbm, pm_hbm, zblk_hbm,
             r_hbm, u_hbm, d_hbm, rbf_hbm, phi_hbm, sph_hbm, chi_hbm,
             idxi_v, idxj_v, pm_v, ri_v, rj_v,
             r_v, u_v, d_v, rbf_v, phi_v, sph_v, w_v, chi_sh, sem):
    cid = lax.axis_index("c")
    sid = lax.axis_index("s")
    wid = cid * NS + sid

    nchunks = N_NODES // B
    full_rounds = nchunks // NS
    pltpu.sync_copy(zblk_hbm, w_v)
    for t in range(full_rounds):
        r0 = pl.multiple_of((sid + NS * t) * B, B)
        pltpu.sync_copy(w_v, chi_sh.at[pl.ds(r0, B)])
    tail_cidx = sid + NS * full_rounds

    @pl.when(tail_cidx < nchunks)
    def _zero_tail():
        r0 = pl.multiple_of(tail_cidx * B, B)
        pltpu.sync_copy(w_v, chi_sh.at[pl.ds(r0, B)])

    plsc.subcore_barrier()

    iota = lax.iota(jnp.int32, L)

    def do_block(blk, _):
        base = wid * PPW + blk * B

        pltpu.sync_copy(idxi_hbm.at[pl.ds(base, B)], idxi_v)
        pltpu.sync_copy(idxj_hbm.at[pl.ds(base, B)], idxj_v)
        pltpu.sync_copy(pm_hbm.at[pl.ds(base, B)], pm_v)

        cp1 = pltpu.async_copy(rpad_hbm.at[idxi_v], ri_v, sem)
        cp2 = pltpu.async_copy(rpad_hbm.at[idxj_v], rj_v, sem)
        cp1.wait()
        cp2.wait()

        cols = [jnp.full((L,), k, jnp.int32) for k in range(N_RBF)]

        def do_group(g, _):
            p = g * L + iota
            off = g * L

            xi = plsc.load_gather(ri_v, [p, cols[0]])
            yi = plsc.load_gather(ri_v, [p, cols[1]])
            zi = plsc.load_gather(ri_v, [p, cols[2]])
            xj = plsc.load_gather(rj_v, [p, cols[0]])
            yj = plsc.load_gather(rj_v, [p, cols[1]])
            zj = plsc.load_gather(rj_v, [p, cols[2]])
            pm = pm_v[pl.ds(off, L)]

            rx = (xj - xi) * pm
            ry = (yj - yi) * pm
            rz = (zj - zi) * pm

            d2 = rx * rx + ry * ry + rz * rz
            safe = d2 > 0.0
            d2s = jnp.where(safe, d2, 1.0)
            bits = lax.bitcast_convert_type(d2s, jnp.int32)
            y = lax.bitcast_convert_type(
                _RSQRT_MAGIC - lax.shift_right_logical(bits, 1), jnp.float32)
            for _i in range(3):
                y = y * (1.5 - 0.5 * d2s * y * y)
            ds_ = d2s * y
            d = jnp.where(safe, ds_, 0.0) * pm

            plsc.store_scatter(r_v, [p, cols[0]], rx)
            plsc.store_scatter(r_v, [p, cols[1]], ry)
            plsc.store_scatter(r_v, [p, cols[2]], rz)
            d_v[pl.ds(off, L)] = d

            for k in range(N_RBF):
                t = d - float(_centers[k])
                e = jnp.exp(_NEG_GAMMA * (t * t)) * pm
                plsc.store_scatter(rbf_v, [p, cols[k]], e)

            s = d * np.float32(1.0 / R_CUT)
            acc = jnp.full((L,), float(_COS_COEF[0]), jnp.float32)
            for ck in _COS_COEF[1:]:
                acc = acc * s + float(ck)
            phi = jnp.where(d < R_CUT, acc, 0.0) * pm
            phi_v[pl.ds(off, L)] = phi

            nz = d != 0.0
            inv = 1.0 / jnp.where(nz, d, 1.0)
            ux = jnp.where(nz, rx * inv, 0.0) * pm
            uy = jnp.where(nz, ry * inv, 0.0) * pm
            uz = jnp.where(nz, rz * inv, 0.0) * pm
            plsc.store_scatter(u_v, [p, cols[0]], ux)
            plsc.store_scatter(u_v, [p, cols[1]], uy)
            plsc.store_scatter(u_v, [p, cols[2]], uz)

            s0 = _SPH_C * uy * pm
            s1 = _SPH_C * uz * pm
            s2 = _SPH_C * ux * pm
            s3 = _L2C1 * ux * uy * pm
            s4 = _L2C1 * uy * uz * pm
            s5 = _L2C2 * (3.0 * uz * uz - 1.0) * pm
            s6 = _L2C1 * ux * uz * pm
            s7 = _L2C3 * (ux * ux - uy * uy) * pm

            for k, sv in enumerate((s0, s1, s2, s3, s4, s5, s6, s7)):
                plsc.store_scatter(sph_v, [p, cols[k]], sv)
                plsc.store_scatter(w_v, [p, cols[k]], sv * phi)
            return 0

        lax.fori_loop(0, G, do_group, 0, unroll=False)

        pltpu.sync_copy(r_v, r_hbm.at[pl.ds(base, B)])
        pltpu.sync_copy(u_v, u_hbm.at[pl.ds(base, B)])
        pltpu.sync_copy(d_v, d_hbm.at[pl.ds(base, B)])
        pltpu.sync_copy(rbf_v, rbf_hbm.at[pl.ds(base, B)])
        pltpu.sync_copy(phi_v, phi_hbm.at[pl.ds(base, B)])
        pltpu.sync_copy(sph_v, sph_hbm.at[pl.ds(base, B)])

        pltpu.sync_copy(w_v, chi_sh.at[idxi_v], add=True)
        return 0

    lax.fori_loop(0, NBLK, do_block, 0, unroll=False)

    plsc.subcore_barrier()

    for t in range(full_rounds):
        r0 = pl.multiple_of((sid + NS * t) * B, B)
        pltpu.sync_copy(chi_sh.at[pl.ds(r0, B)], w_v)
        pltpu.sync_copy(w_v, chi_hbm.at[cid, pl.ds(r0, B)])

    @pl.when(tail_cidx < nchunks)
    def _write_tail():
        r0 = pl.multiple_of(tail_cidx * B, B)
        pltpu.sync_copy(chi_sh.at[pl.ds(r0, B)], w_v)
        pltpu.sync_copy(w_v, chi_hbm.at[cid, pl.ds(r0, B)])


_sc_call = pl.kernel(
    _sc_body,
    out_type=(
        jax.ShapeDtypeStruct((N_PAIRS, 3), jnp.float32),
        jax.ShapeDtypeStruct((N_PAIRS, 3), jnp.float32),
        jax.ShapeDtypeStruct((N_PAIRS,), jnp.float32),
        jax.ShapeDtypeStruct((N_PAIRS, 16), jnp.float32),
        jax.ShapeDtypeStruct((N_PAIRS,), jnp.float32),
        jax.ShapeDtypeStruct((N_PAIRS, 8), jnp.float32),
        jax.ShapeDtypeStruct((NC, N_NODES, 8), jnp.float32),
    ),
    mesh=plsc.VectorSubcoreMesh(core_axis_name="c", subcore_axis_name="s"),
    compiler_params=pltpu.CompilerParams(
        needs_layout_passes=False, use_tc_tiling_on_sc=False),
    scratch_types=(
        pltpu.VMEM((B,), jnp.int32),
        pltpu.VMEM((B,), jnp.int32),
        pltpu.VMEM((B,), jnp.float32),
        pltpu.VMEM((B, 8), jnp.float32),
        pltpu.VMEM((B, 8), jnp.float32),
        pltpu.VMEM((B, 3), jnp.float32),
        pltpu.VMEM((B, 3), jnp.float32),
        pltpu.VMEM((B,), jnp.float32),
        pltpu.VMEM((B, 16), jnp.float32),
        pltpu.VMEM((B,), jnp.float32),
        pltpu.VMEM((B, 8), jnp.float32),
        pltpu.VMEM((B, 8), jnp.float32),
        pltpu.VMEM_SHARED((N_NODES, 8), jnp.float32),
        pltpu.SemaphoreType.DMA,
    ),
)


def _combine_body(p_ref, m_ref, o_ref):
    o_ref[...] = (p_ref[0] + p_ref[1]) * m_ref[...] * np.float32(1.0 / LAMBDA)


_combine = pl.pallas_call(
    _combine_body,
    out_shape=jax.ShapeDtypeStruct((N_NODES * 8 // 128, 128), jnp.float32),
)


def kernel(R, idx_i, idx_j, pair_mask, z, point_mask):
    del z
    rpad = jnp.zeros((N_NODES, 8), jnp.float32).at[:, :3].set(R)
    zblk = jnp.zeros((B, 8), jnp.float32)
    r_ij, u_ij, d_ij, rbf_ij, phi, sph_ij, chi_parts = _sc_call(
        rpad, idx_i, idx_j, pair_mask, zblk)
    mask_rep = jnp.repeat(point_mask, 8).reshape(N_NODES * 8 // 128, 128)
    chi = _combine(chi_parts.reshape(NC, N_NODES * 8 // 128, 128), mask_rep)
    return (r_ij, u_ij, d_ij, rbf_ij, phi, sph_ij, chi.reshape(N_NODES, 8))


# speedup vs baseline: 23.2683x; 4.0136x over previous
import jax
import jax.numpy as jnp
import numpy as np
from jax import lax
from jax.experimental import pallas as pl
from jax.experimental.pallas import tpu as pltpu
from jax.experimental.pallas import tpu_sc as plsc

N_NODES = 100000
N_PAIRS = 3200000
R_CUT = 5.0
N_RBF = 16
LAMBDA = 1.0

NC = 2
NS = 16
L = 16
NW = NC * NS
B = 1024
G = B // L
TCOL = B // 128
NBLK = N_PAIRS // B
NBODY = (NBLK + NW - 1) // NW

_centers = np.linspace(0.0, R_CUT, N_RBF).astype(np.float32)
_delta = np.float32(_centers[1] - _centers[0])
_GAMMA = np.float32(np.float32(0.5) / (_delta * _delta))
_NEG_GAMMA = np.float32(-_GAMMA)

_xs = 0.5 * (1.0 + np.cos(np.pi * (np.arange(512) + 0.5) / 512.0))
_ys = 0.5 * (1.0 + np.cos(np.pi * _xs))
_COS_COEF = np.polyfit(_xs, _ys, 12).astype(np.float32)

_SPH_C = np.float32(0.4886025119029199)
_L2C1 = np.float32(1.0925484305920792)
_L2C2 = np.float32(0.31539156525252005)
_L2C3 = np.float32(0.5462742152960396)

_RSQRT_MAGIC = np.int32(0x5F3759DF)


def _sc_body(*refs):
    (rpad_hbm, idxi_hbm, idxj_hbm, pm_hbm, zblk_hbm,
     r_hbm, u_hbm, rbf_hbm, sph_hbm, chi_hbm) = refs[:10]
    sc = list(refs[10:])
    idxi_v = [sc.pop(0)]
    idxj_v = [sc.pop(0)]
    pm_v = [sc.pop(0)]
    ri_v = [sc.pop(0)]
    rj_v = [sc.pop(0)]
    r_v = [sc.pop(0)]
    u_v = [sc.pop(0)]
    rbf_v = [sc.pop(0)]
    sph_v = [sc.pop(0)]
    w_v = [sc.pop(0)]
    chi_sh = sc.pop(0)
    insem = [sc.pop(0)]
    gsem = [sc.pop(0)]
    osem = [sc.pop(0)]
    assert not sc

    cid = lax.axis_index("c")
    sid = lax.axis_index("s")
    wid = cid * NS + sid

    pltpu.sync_copy(zblk_hbm, w_v[0])
    nfull = N_NODES // B
    for t in range(7):
        c = sid + NS * t

        @pl.when(c < nfull)
        def _z():
            pltpu.sync_copy(w_v[0], chi_sh.at[pl.ds(pl.multiple_of(c * B, B), B)])

    @pl.when(sid == 15)
    def _ztail():
        pltpu.sync_copy(w_v[0].at[pl.ds(0, N_NODES - nfull * B)],
                        chi_sh.at[pl.ds(nfull * B, N_NODES - nfull * B)])

    plsc.subcore_barrier()

    iota = lax.iota(jnp.int32, L)
    cols = [jnp.full((L,), k, jnp.int32) for k in range(8)]

    def issue_in(blk, s):
        base = pl.multiple_of(blk * B, B)
        sem = insem[0]
        pltpu.async_copy(idxi_hbm.at[pl.ds(base, B)], idxi_v[s], sem).wait()
        pltpu.async_copy(idxj_hbm.at[pl.ds(base, B)], idxj_v[s], sem).wait()
        pltpu.async_copy(pm_hbm.at[pl.ds(base, B)], pm_v[s], sem).wait()

    def issue_gather(s):
        pltpu.async_copy(rpad_hbm.at[idxi_v[s]], ri_v[s], gsem[0]).wait()
        pltpu.async_copy(rpad_hbm.at[idxj_v[s]], rj_v[s], gsem[0]).wait()

    def issue_out(blk, s):
        base = pl.multiple_of(blk * B, B)
        b512 = pl.multiple_of(blk * (TCOL * 512), TCOL * 512)
        b1024 = pl.multiple_of(blk * (TCOL * 1024), TCOL * 1024)
        b1024h = pl.multiple_of(NBLK * TCOL * 1024 + blk * (TCOL * 1024), TCOL * 1024)
        sem = osem[0]
        pltpu.async_copy(r_v[0], r_hbm.at[pl.ds(b512, TCOL * 512)], sem).wait()
        pltpu.async_copy(u_v[0], u_hbm.at[pl.ds(b512, TCOL * 512)], sem).wait()
        pltpu.async_copy(
            rbf_v[0].at[pl.ds(0, TCOL * 1024)],
            rbf_hbm.at[pl.ds(b1024, TCOL * 1024)], sem).wait()
        pltpu.async_copy(
            rbf_v[0].at[pl.ds(TCOL * 1024, TCOL * 1024)],
            rbf_hbm.at[pl.ds(b1024h, TCOL * 1024)], sem).wait()
        pltpu.async_copy(sph_v[0], sph_hbm.at[pl.ds(b1024, TCOL * 1024)], sem).wait()
        pltpu.async_copy(w_v[0], chi_sh.at[idxi_v[s]], sem, add=True).wait()

    def compute(s):
        def do_group(g, _):
            q0 = g * L
            p = q0 + iota
            tc = q0 // 128
            l0 = q0 % 128
            lv512 = tc * 512 + l0 + iota
            lv1024 = tc * 1024 + l0 + iota

            xi = plsc.load_gather(ri_v[s], [p, cols[0]])
            yi = plsc.load_gather(ri_v[s], [p, cols[1]])
            zi = plsc.load_gather(ri_v[s], [p, cols[2]])
            xj = plsc.load_gather(rj_v[s], [p, cols[0]])
            yj = plsc.load_gather(rj_v[s], [p, cols[1]])
            zj = plsc.load_gather(rj_v[s], [p, cols[2]])
            pm = pm_v[s][pl.ds(q0, L)]

            rx = (xj - xi) * pm
            ry = (yj - yi) * pm
            rz = (zj - zi) * pm

            d2 = rx * rx + ry * ry + rz * rz
            safe = d2 > 0.0
            d2s = jnp.where(safe, d2, 1.0)
            bits = lax.bitcast_convert_type(d2s, jnp.int32)
            y = lax.bitcast_convert_type(
                _RSQRT_MAGIC - lax.shift_right_logical(bits, 1), jnp.float32)
            for _i in range(3):
                y = y * (1.5 - 0.5 * d2s * y * y)
            ds_ = d2s * y
            d = jnp.where(safe, ds_, 0.0) * pm

            plsc.store_scatter(r_v[0], [lv512], rx)
            plsc.store_scatter(r_v[0], [lv512 + 128], ry)
            plsc.store_scatter(r_v[0], [lv512 + 256], rz)
            plsc.store_scatter(r_v[0], [lv512 + 384], d)

            for k in range(N_RBF):
                tt = d - float(_centers[k])
                e = jnp.exp(_NEG_GAMMA * (tt * tt)) * pm
                if k < 8:
                    plsc.store_scatter(rbf_v[0], [lv1024 + k * 128], e)
                else:
                    plsc.store_scatter(
                        rbf_v[0], [TCOL * 1024 + lv1024 + (k - 8) * 128], e)

            sca = d * np.float32(1.0 / R_CUT)
            acc = jnp.full((L,), float(_COS_COEF[0]), jnp.float32)
            for ck in _COS_COEF[1:]:
                acc = acc * sca + float(ck)
            phi = jnp.where(d < R_CUT, acc, 0.0) * pm

            nz = d != 0.0
            inv = 1.0 / jnp.where(nz, d, 1.0)
            ux = jnp.where(nz, rx * inv, 0.0) * pm
            uy = jnp.where(nz, ry * inv, 0.0) * pm
            uz = jnp.where(nz, rz * inv, 0.0) * pm
            plsc.store_scatter(u_v[0], [lv512], ux)
            plsc.store_scatter(u_v[0], [lv512 + 128], uy)
            plsc.store_scatter(u_v[0], [lv512 + 256], uz)
            plsc.store_scatter(u_v[0], [lv512 + 384], phi)

            s0 = _SPH_C * uy * pm
            s1 = _SPH_C * uz * pm
            s2 = _SPH_C * ux * pm
            s3 = _L2C1 * ux * uy * pm
            s4 = _L2C1 * uy * uz * pm
            s5 = _L2C2 * (3.0 * uz * uz - 1.0) * pm
            s6 = _L2C1 * ux * uz * pm
            s7 = _L2C3 * (ux * ux - uy * uy) * pm

            for k, sv in enumerate((s0, s1, s2, s3, s4, s5, s6, s7)):
                plsc.store_scatter(sph_v[0], [lv1024 + k * 128], sv)
                plsc.store_scatter(w_v[0], [p, cols[k]], sv * phi)
            return 0

        lax.fori_loop(0, G, do_group, 0, unroll=False)

    def body(i, _):
        blk = wid + NW * i

        @pl.when(blk < NBLK)
        def _go():
            issue_in(blk, 0)
            issue_gather(0)
            compute(0)
            issue_out(blk, 0)

        return 0

    lax.fori_loop(0, NBODY, body, 0, unroll=False)

    plsc.subcore_barrier()

    for t in range(7):
        c = sid + NS * t

        @pl.when(c < nfull)
        def _wb():
            r0 = pl.multiple_of(c * B, B)
            pltpu.sync_copy(chi_sh.at[pl.ds(r0, B)], w_v[0])
            pltpu.sync_copy(w_v[0], chi_hbm.at[cid, pl.ds(r0, B)])

    @pl.when(sid == 15)
    def _wbtail():
        rem = N_NODES - nfull * B
        pltpu.sync_copy(chi_sh.at[pl.ds(nfull * B, rem)], w_v[0].at[pl.ds(0, rem)])
        pltpu.sync_copy(w_v[0].at[pl.ds(0, rem)], chi_hbm.at[cid, pl.ds(nfull * B, rem)])


_scratch = (
    [pltpu.VMEM((B,), jnp.int32)]
    + [pltpu.VMEM((B,), jnp.int32)]
    + [pltpu.VMEM((B,), jnp.float32)]
    + [pltpu.VMEM((B, 8), jnp.float32)]
    + [pltpu.VMEM((B, 8), jnp.float32)]
    + [pltpu.VMEM((TCOL * 512,), jnp.float32)]
    + [pltpu.VMEM((TCOL * 512,), jnp.float32)]
    + [pltpu.VMEM((TCOL * 2048,), jnp.float32)]
    + [pltpu.VMEM((TCOL * 1024,), jnp.float32)]
    + [pltpu.VMEM((B, 8), jnp.float32)]
    + [pltpu.VMEM_SHARED((N_NODES, 8), jnp.float32)]
    + [pltpu.SemaphoreType.DMA] * 3
)

_sc_call = pl.kernel(
    _sc_body,
    out_type=(
        jax.ShapeDtypeStruct((N_PAIRS * 4,), jnp.float32),
        jax.ShapeDtypeStruct((N_PAIRS * 4,), jnp.float32),
        jax.ShapeDtypeStruct((N_PAIRS * 16,), jnp.float32),
        jax.ShapeDtypeStruct((N_PAIRS * 8,), jnp.float32),
        jax.ShapeDtypeStruct((NC, N_NODES, 8), jnp.float32),
    ),
    mesh=plsc.VectorSubcoreMesh(core_axis_name="c", subcore_axis_name="s"),
    compiler_params=pltpu.CompilerParams(
        needs_layout_passes=False, use_tc_tiling_on_sc=False),
    scratch_types=tuple(_scratch),
)


def _combine_body(p_ref, m_ref, o_ref):
    o_ref[...] = (p_ref[0] + p_ref[1]) * m_ref[...] * np.float32(1.0 / LAMBDA)


_combine = pl.pallas_call(
    _combine_body,
    out_shape=jax.ShapeDtypeStruct((N_NODES * 8 // 128, 128), jnp.float32),
)

_NT = N_PAIRS // 128


def kernel(R, idx_i, idx_j, pair_mask, z, point_mask):
    del z
    rpad = jnp.pad(R, ((0, 0), (0, 5)))
    zblk = jnp.zeros((B, 8), jnp.float32)
    r_f, u_f, rbf_f, sph_f, chi_parts = _sc_call(
        rpad, idx_i, idx_j, pair_mask, zblk)
    mask_rep = jnp.repeat(point_mask, 8).reshape(N_NODES * 8 // 128, 128)
    chi = _combine(chi_parts.reshape(NC, N_NODES * 8 // 128, 128), mask_rep)
    r4 = r_f.reshape(_NT, 4, 128)
    u4 = u_f.reshape(_NT, 4, 128)
    r_ij = r4[:, :3, :].transpose(0, 2, 1).reshape(N_PAIRS, 3)
    u_ij = u4[:, :3, :].transpose(0, 2, 1).reshape(N_PAIRS, 3)
    d_ij = r4[:, 3, :].reshape(N_PAIRS)
    phi = u4[:, 3, :].reshape(N_PAIRS)
    rbf_ij = rbf_f.reshape(2, _NT, 8, 128).transpose(1, 3, 0, 2).reshape(N_PAIRS, 16)
    sph_ij = sph_f.reshape(_NT, 8, 128).transpose(0, 2, 1).reshape(N_PAIRS, 8)
    return (r_ij, u_ij, d_ij, rbf_ij, phi, sph_ij, chi.reshape(N_NODES, 8))
